# baseline (speedup 1.0000x reference)
import functools

import numpy as np

import jax
import jax.numpy as jnp
from jax import lax
from jax.experimental import pallas as pl
from jax.experimental.pallas import tpu as pltpu
from jax.experimental.pallas import tpu_sc as plsc

N = 10000
D = 128
Z = 64
E = 320000
NPAD = 10240
NC = 2
NS = 16
NT = NC * NS
K = 128
CH = 81
EPT = K * CH
E_PAD = NT * EPT
RPT = NPAD // NS
R = 1024
NBLK = NPAD // R
NSP = 10112
RPTS = NSP // NS

_f32 = jnp.float32


def _mesh():
    return plsc.VectorSubcoreMesh(core_axis_name="c", subcore_axis_name="s")


_EPS_CACHE = []


def _eps_pad():
    if not _EPS_CACHE:
        try:
            with jax.ensure_compile_time_eval():
                e = np.asarray(
                    jax.random.normal(jax.random.key(42), (N, Z), _f32))
            ep = np.zeros((NPAD, Z), np.float32)
            ep[:N] = e
        except Exception:
            ep = None
        _EPS_CACHE.append(ep)
    if _EPS_CACHE[0] is None:
        return jnp.pad(jax.random.normal(jax.random.key(42), (N, Z), _f32),
                       ((0, NPAD - N), (0, 0)))
    return _EPS_CACHE[0]


def _deg_body(src, dst, zeros16, ones, out, idxs, idxd, ones_v, d_out, d_in,
              s0, s1):
    c = lax.axis_index("c")
    s = lax.axis_index("s")
    tid = c * NS + s
    pltpu.sync_copy(zeros16, d_out.at[pl.ds(s * RPT, RPT)])
    pltpu.sync_copy(zeros16, d_in.at[pl.ds(s * RPT, RPT)])
    pltpu.sync_copy(ones, ones_v)
    pltpu.sync_copy(src.at[tid], idxs)
    pltpu.sync_copy(dst.at[tid], idxd)
    plsc.subcore_barrier()

    def body(j, carry):
        pltpu.async_copy(ones_v, d_out.at[idxs.at[j]], s0, add=True)
        pltpu.async_copy(ones_v, d_in.at[idxd.at[j]], s1, add=True)
        return carry

    lax.fori_loop(0, CH, body, 0)

    def drain(j, carry):
        pltpu.make_async_copy(ones_v, d_out.at[idxs.at[j]], s0).wait()
        pltpu.make_async_copy(ones_v, d_in.at[idxd.at[j]], s1).wait()
        return carry

    lax.fori_loop(0, CH, drain, 0)
    plsc.subcore_barrier()
    base = (c * 2) * NPAD
    pltpu.sync_copy(d_out.at[pl.ds(s * RPT, RPT)],
                    out.at[pl.ds(base + s * RPT, RPT)])
    pltpu.sync_copy(d_in.at[pl.ds(s * RPT, RPT)],
                    out.at[pl.ds(base + NPAD + s * RPT, RPT)])


@functools.cache
def _deg_kernel():
    return pl.kernel(
        _deg_body,
        out_type=jax.ShapeDtypeStruct((NC * 2 * NPAD, 16), _f32),
        mesh=_mesh(),
        scratch_types=[
            pltpu.VMEM((CH, K), jnp.int32),
            pltpu.VMEM((CH, K), jnp.int32),
            pltpu.VMEM((K, 16), _f32),
            pltpu.VMEM_SHARED((NPAD, 16), _f32),
            pltpu.VMEM_SHARED((NPAD, 16), _f32),
            pltpu.SemaphoreType.DMA,
            pltpu.SemaphoreType.DMA,
        ],
    )


def _deg(src3, dst3, zeros16, ones):
    return _deg_kernel()(src3, dst3, zeros16, ones)


def _spmm_body(hw, sd, zeros, out, idx, rows, agg, g0, g1, g2, i0, i1, i2):
    gsem = (g0, g1, g2)
    isem = (i0, i1, i2)
    c = lax.axis_index("c")
    s = lax.axis_index("s")
    base = (c * NS + s) * CH
    pltpu.sync_copy(zeros, agg.at[pl.ds(s * RPTS, RPTS)])
    plsc.subcore_barrier()

    for b in range(3):
        pltpu.async_copy(sd.at[base + b], idx.at[b], isem[b])
    for b in range(2):
        pltpu.make_async_copy(sd.at[base + b], idx.at[b], isem[b]).wait()
        pltpu.async_copy(hw.at[idx.at[b, 0]], rows.at[b], gsem[b])

    def slot(j, b0, b2):
        jn2 = j + 2
        jn3 = j + 3

        @pl.when(jn2 < CH)
        def _():
            pltpu.make_async_copy(sd.at[base + jn2], idx.at[b2],
                                  isem[b2]).wait()
            pltpu.async_copy(hw.at[idx.at[b2, 0]], rows.at[b2], gsem[b2])

        pltpu.make_async_copy(hw.at[idx.at[b0, 0]], rows.at[b0],
                              gsem[b0]).wait()
        pltpu.sync_copy(rows.at[b0], agg.at[idx.at[b0, 1]], add=True)

        @pl.when(jn3 < CH)
        def _():
            pltpu.async_copy(sd.at[base + jn3], idx.at[b0], isem[b0])

    def round_body(g, carry):
        slot(3 * g, 0, 2)
        slot(3 * g + 1, 1, 0)
        slot(3 * g + 2, 2, 1)
        return carry

    lax.fori_loop(0, CH // 3, round_body, 0)
    plsc.subcore_barrier()
    pltpu.sync_copy(agg.at[pl.ds(s * RPTS, RPTS)],
                    out.at[pl.ds(c * NPAD + s * RPTS, RPTS)])


@functools.cache
def _spmm_kernel():
    return pl.kernel(
        _spmm_body,
        out_type=jax.ShapeDtypeStruct((NC * NPAD, D), _f32),
        mesh=_mesh(),
        scratch_types=[
            pltpu.VMEM((3, 2, K), jnp.int32),
            pltpu.VMEM((3, K, D), _f32),
            pltpu.VMEM_SHARED((NSP, D), _f32),
        ] + [pltpu.SemaphoreType.DMA] * 6,
    )


def _spmm(hw, sd3, zeros):
    return _spmm_kernel()(hw, sd3, zeros)


def _norm_col(a, b):
    d = a[:, 0:1] + b[:, 0:1]
    return jnp.where(d > 0, lax.rsqrt(jnp.maximum(d, 1e-12)), 0.0)


def _mm(x, w):
    return jnp.dot(x, w, preferred_element_type=_f32,
                   precision=lax.Precision.HIGHEST)


def _agg_sum(p0, p1):
    return p0[...] + p1[...]


def _pro1_body(feat, dgo0, dgo1, w, out):
    nsrc = _norm_col(dgo0[...], dgo1[...])
    out[...] = _mm(feat[...] * nsrc, w[...])


def _eppro_body(p0, p1, dgi0, dgi1, dgo0, dgo1, b, w, out):
    ndst = _norm_col(dgi0[...], dgi1[...])
    h = jnp.maximum(_agg_sum(p0, p1) * ndst + b[...][None, :], 0.0)
    nsrc = _norm_col(dgo0[...], dgo1[...])
    out[...] = _mm(h * nsrc, w[...])


def _mid_body(p0, p1, dgi0, dgi1, dgo0, dgo1, b2, wrep, brep, wrec, brec,
              wd1, eps, hw3, kl):
    i = pl.program_id(0)
    ndst = _norm_col(dgi0[...], dgi1[...])
    emb = _agg_sum(p0, p1) * ndst + b2[...][None, :]
    mu = _mm(emb, wrep[...]) + brep[...][None, :]
    expmu = jnp.exp(mu)
    z = mu + eps[...] * jnp.exp(mu * 0.5)
    hz = _mm(z, wrec[...]) + brec[...][None, :]
    nsrc = _norm_col(dgo0[...], dgo1[...])
    hw3[...] = _mm(hz * nsrc, wd1[...])
    rows = i * R + lax.broadcasted_iota(jnp.int32, (R, 1), 0)
    mask = jnp.broadcast_to(rows < N, mu.shape)
    klb = -0.5 * jnp.sum(jnp.where(mask, 1.0 + mu - mu * mu - expmu, 0.0))

    @pl.when(i == 0)
    def _():
        kl[0, 0] = 0.0

    kl[0, 0] += klb


def _final_body(p0, p1, dgi0, dgi1, b, feat, klin, loss):
    i = pl.program_id(0)
    ndst = _norm_col(dgi0[...], dgi1[...])
    xrec = _agg_sum(p0, p1) * ndst + b[...][None, :]
    diff = xrec - feat[...]
    rows = i * R + lax.broadcasted_iota(jnp.int32, (R, 1), 0)
    mask = jnp.broadcast_to(rows < N, diff.shape)
    s = jnp.sum(jnp.where(mask, diff * diff, 0.0))

    @pl.when(i == 0)
    def _():
        loss[0, 0] = klin[0, 0]

    loss[0, 0] += s


def _rowblk(nc, dt=_f32):
    return pl.BlockSpec((R, nc), lambda i: (i, 0))


def _degblk(part):
    return pl.BlockSpec((R, 16), lambda i, p=part: (p * NBLK + i, 0))


def _wblk(r, c):
    return pl.BlockSpec((r, c), lambda i: (0, 0))


def _bblk(nc):
    return pl.BlockSpec((nc,), lambda i: (0,))


def _sblk():
    return pl.BlockSpec((1, 1), lambda i: (0, 0), memory_space=pltpu.SMEM)


def _p_specs():
    return [pl.BlockSpec((R, D), lambda i: (i, 0)),
            pl.BlockSpec((R, D), lambda i: (NBLK + i, 0))]


def _hw_shape():
    return jax.ShapeDtypeStruct((NPAD, D), _f32)


def _pro1_call(feat, deg, w):
    return pl.pallas_call(
        _pro1_body,
        grid=(NBLK,),
        in_specs=[_rowblk(D), _degblk(0), _degblk(2), _wblk(D, D)],
        out_specs=_rowblk(D),
        out_shape=_hw_shape(),
    )(feat, deg, deg, w)


def _eppro_call(p, deg, b, w):
    return pl.pallas_call(
        _eppro_body,
        grid=(NBLK,),
        in_specs=_p_specs() + [_degblk(1), _degblk(3), _degblk(0), _degblk(2),
                               _bblk(D), _wblk(D, D)],
        out_specs=_rowblk(D),
        out_shape=_hw_shape(),
    )(p, p, deg, deg, deg, deg, b, w)


def _mid_call(p, deg, b2, wrep, brep, wrec, brec, wd1, eps):
    return pl.pallas_call(
        _mid_body,
        grid=(NBLK,),
        in_specs=_p_specs() + [_degblk(1), _degblk(3), _degblk(0), _degblk(2),
                               _bblk(D), _wblk(D, Z), _bblk(Z), _wblk(Z, D),
                               _bblk(D), _wblk(D, D), _rowblk(Z)],
        out_specs=[_rowblk(D), _sblk()],
        out_shape=[_hw_shape(), jax.ShapeDtypeStruct((1, 1), _f32)],
    )(p, p, deg, deg, deg, deg, b2, wrep, brep, wrec, brec, wd1, eps)


def _final_call(p, deg, b, feat, kl):
    return pl.pallas_call(
        _final_body,
        grid=(NBLK,),
        in_specs=_p_specs() + [_degblk(1), _degblk(3), _bblk(D), _rowblk(D),
                               _sblk()],
        out_specs=_sblk(),
        out_shape=jax.ShapeDtypeStruct((1, 1), _f32),
    )(p, p, deg, deg, b, feat, kl)


def kernel(features, edge_index, enc_W1, enc_b1, enc_W2, enc_b2, W_rep, b_rep,
           W_rec, b_rec, dec_W1, dec_b1, dec_W2, dec_b2):
    pad_idx = N + jnp.arange(E_PAD - E, dtype=jnp.int32) % (NSP - N)
    ei_pad = jnp.concatenate(
        [edge_index, jnp.broadcast_to(pad_idx, (2, E_PAD - E))], axis=1)
    ei3 = ei_pad.reshape(2, NT * CH, K)
    sd3 = ei3.transpose(1, 0, 2)
    feat = jnp.pad(features, ((0, NPAD - N), (0, 0)))
    eps = jnp.asarray(_eps_pad())
    zeros = jnp.zeros((RPTS, D), _f32)
    zeros16 = jnp.zeros((RPT, 16), _f32)
    ones = jnp.ones((K, 16), _f32)

    deg = _deg(ei3[0].reshape(NT, CH, K), ei3[1].reshape(NT, CH, K),
               zeros16, ones)
    hw1 = _pro1_call(feat, deg, enc_W1)
    p = _spmm(hw1, sd3, zeros)
    hw2 = _eppro_call(p, deg, enc_b1, enc_W2)
    p = _spmm(hw2, sd3, zeros)
    hw3, kl = _mid_call(p, deg, enc_b2, W_rep, b_rep, W_rec, b_rec, dec_W1, eps)
    p = _spmm(hw3, sd3, zeros)
    hw4 = _eppro_call(p, deg, dec_b1, dec_W2)
    p = _spmm(hw4, sd3, zeros)
    loss = _final_call(p, deg, dec_b2, feat, kl)
    return loss[0, 0]


# speedup vs baseline: 1.0240x; 1.0240x over previous
import functools

import numpy as np

import jax
import jax.numpy as jnp
from jax import lax
from jax.experimental import pallas as pl
from jax.experimental.pallas import tpu as pltpu
from jax.experimental.pallas import tpu_sc as plsc

N = 10000
D = 128
Z = 64
E = 320000
NPAD = 10240
NC = 2
NS = 16
NT = NC * NS
K = 128
CH = 81
EPT = K * CH
E_PAD = NT * EPT
RPT = NPAD // NS
R = 1024
NBLK = NPAD // R
NSP = 10112
RPTS = NSP // NS

_f32 = jnp.float32


def _mesh():
    return plsc.VectorSubcoreMesh(core_axis_name="c", subcore_axis_name="s")


_EPS_CACHE = []


def _eps_pad():
    if not _EPS_CACHE:
        try:
            with jax.ensure_compile_time_eval():
                e = np.asarray(
                    jax.random.normal(jax.random.key(42), (N, Z), _f32))
            ep = np.zeros((NPAD, Z), np.float32)
            ep[:N] = e
        except Exception:
            ep = None
        _EPS_CACHE.append(ep)
    if _EPS_CACHE[0] is None:
        return jnp.pad(jax.random.normal(jax.random.key(42), (N, Z), _f32),
                       ((0, NPAD - N), (0, 0)))
    return _EPS_CACHE[0]


def _deg_body(src, dst, zeros16, ones, out, idxs, idxd, ones_v, d_out, d_in,
              s0, s1):
    c = lax.axis_index("c")
    s = lax.axis_index("s")
    tid = c * NS + s
    pltpu.sync_copy(zeros16, d_out.at[pl.ds(s * RPT, RPT)])
    pltpu.sync_copy(zeros16, d_in.at[pl.ds(s * RPT, RPT)])
    pltpu.sync_copy(ones, ones_v)
    pltpu.sync_copy(src.at[tid], idxs)
    pltpu.sync_copy(dst.at[tid], idxd)
    plsc.subcore_barrier()

    def body(j, carry):
        pltpu.async_copy(ones_v, d_out.at[idxs.at[j]], s0, add=True)
        pltpu.async_copy(ones_v, d_in.at[idxd.at[j]], s1, add=True)
        return carry

    lax.fori_loop(0, CH, body, 0)

    def drain(j, carry):
        pltpu.make_async_copy(ones_v, d_out.at[idxs.at[j]], s0).wait()
        pltpu.make_async_copy(ones_v, d_in.at[idxd.at[j]], s1).wait()
        return carry

    lax.fori_loop(0, CH, drain, 0)
    plsc.subcore_barrier()
    base = (c * 2) * NPAD
    pltpu.sync_copy(d_out.at[pl.ds(s * RPT, RPT)],
                    out.at[pl.ds(base + s * RPT, RPT)])
    pltpu.sync_copy(d_in.at[pl.ds(s * RPT, RPT)],
                    out.at[pl.ds(base + NPAD + s * RPT, RPT)])


@functools.cache
def _deg_kernel():
    return pl.kernel(
        _deg_body,
        out_type=jax.ShapeDtypeStruct((NC * 2 * NPAD, 16), _f32),
        mesh=_mesh(),
        scratch_types=[
            pltpu.VMEM((CH, K), jnp.int32),
            pltpu.VMEM((CH, K), jnp.int32),
            pltpu.VMEM((K, 16), _f32),
            pltpu.VMEM_SHARED((NPAD, 16), _f32),
            pltpu.VMEM_SHARED((NPAD, 16), _f32),
            pltpu.SemaphoreType.DMA,
            pltpu.SemaphoreType.DMA,
        ],
    )


def _deg(src3, dst3, zeros16, ones):
    return _deg_kernel()(src3, dst3, zeros16, ones)


def _spmm_body(hw, sd, zeros, out, idx, rows, agg, g0, g1, g2, i0, i1, i2):
    gsem = (g0, g1, g2)
    isem = (i0, i1, i2)
    c = lax.axis_index("c")
    s = lax.axis_index("s")
    base = (c * NS + s) * CH
    pltpu.sync_copy(zeros, agg.at[pl.ds(s * RPTS, RPTS)])
    plsc.subcore_barrier()

    for b in range(3):
        pltpu.async_copy(sd.at[base + b], idx.at[b], isem[b])
    for b in range(2):
        pltpu.make_async_copy(sd.at[base + b], idx.at[b], isem[b]).wait()
        pltpu.async_copy(hw.at[idx.at[b, 0]], rows.at[b], gsem[b])

    def slot(j, b0, b2):
        jn2 = j + 2
        jn3 = j + 3

        @pl.when(jn2 < CH)
        def _():
            pltpu.make_async_copy(sd.at[base + jn2], idx.at[b2],
                                  isem[b2]).wait()
            pltpu.async_copy(hw.at[idx.at[b2, 0]], rows.at[b2], gsem[b2])

        pltpu.make_async_copy(hw.at[idx.at[b0, 0]], rows.at[b0],
                              gsem[b0]).wait()
        pltpu.sync_copy(rows.at[b0], agg.at[idx.at[b0, 1]], add=True)

        @pl.when(jn3 < CH)
        def _():
            pltpu.async_copy(sd.at[base + jn3], idx.at[b0], isem[b0])

    def round_body(g, carry):
        slot(3 * g, 0, 2)
        slot(3 * g + 1, 1, 0)
        slot(3 * g + 2, 2, 1)
        return carry

    lax.fori_loop(0, CH // 3, round_body, 0)
    plsc.subcore_barrier()
    pltpu.sync_copy(agg.at[pl.ds(s * RPTS, RPTS)],
                    out.at[pl.ds(c * NPAD + s * RPTS, RPTS)])


@functools.cache
def _spmm_kernel():
    return pl.kernel(
        _spmm_body,
        out_type=jax.ShapeDtypeStruct((NC * NPAD, D), _f32),
        mesh=_mesh(),
        scratch_types=[
            pltpu.VMEM((3, 2, K), jnp.int32),
            pltpu.VMEM((3, K, D), _f32),
            pltpu.VMEM_SHARED((NSP, D), _f32),
        ] + [pltpu.SemaphoreType.DMA] * 6,
    )


def _spmm(hw, sd3, zeros):
    return _spmm_kernel()(hw, sd3, zeros)


def _norm_col(a, b):
    d = a[:, 0:1] + b[:, 0:1]
    return jnp.where(d > 0, lax.rsqrt(jnp.maximum(d, 1e-12)), 0.0)


def _mm(x, w):
    return jnp.dot(x, w, preferred_element_type=_f32,
                   precision=lax.Precision.HIGHEST)


def _agg_sum(p0, p1):
    return p0[...] + p1[...]


def _pro1_body(feat, dgo0, dgo1, w, out):
    nsrc = _norm_col(dgo0[...], dgo1[...])
    out[...] = _mm(feat[...] * nsrc, w[...])


def _eppro_body(p0, p1, dgi0, dgi1, dgo0, dgo1, b, w, out):
    ndst = _norm_col(dgi0[...], dgi1[...])
    h = jnp.maximum(_agg_sum(p0, p1) * ndst + b[...][None, :], 0.0)
    nsrc = _norm_col(dgo0[...], dgo1[...])
    out[...] = _mm(h * nsrc, w[...])


def _mid_body(p0, p1, dgi0, dgi1, dgo0, dgo1, b2, wrep, brep, wrec, brec,
              wd1, eps, hw3, kl):
    i = pl.program_id(0)
    ndst = _norm_col(dgi0[...], dgi1[...])
    emb = _agg_sum(p0, p1) * ndst + b2[...][None, :]
    mu = _mm(emb, wrep[...]) + brep[...][None, :]
    expmu = jnp.exp(mu)
    z = mu + eps[...] * jnp.exp(mu * 0.5)
    hz = _mm(z, wrec[...]) + brec[...][None, :]
    nsrc = _norm_col(dgo0[...], dgo1[...])
    hw3[...] = _mm(hz * nsrc, wd1[...])
    rows = i * R + lax.broadcasted_iota(jnp.int32, (R, 1), 0)
    mask = jnp.broadcast_to(rows < N, mu.shape)
    klb = -0.5 * jnp.sum(jnp.where(mask, 1.0 + mu - mu * mu - expmu, 0.0))

    @pl.when(i == 0)
    def _():
        kl[0, 0] = 0.0

    kl[0, 0] += klb


def _final_body(p0, p1, dgi0, dgi1, b, feat, klin, loss):
    i = pl.program_id(0)
    ndst = _norm_col(dgi0[...], dgi1[...])
    xrec = _agg_sum(p0, p1) * ndst + b[...][None, :]
    diff = xrec - feat[...]
    rows = i * R + lax.broadcasted_iota(jnp.int32, (R, 1), 0)
    mask = jnp.broadcast_to(rows < N, diff.shape)
    s = jnp.sum(jnp.where(mask, diff * diff, 0.0))

    @pl.when(i == 0)
    def _():
        loss[0, 0] = klin[0, 0]

    loss[0, 0] += s


def _rowblk(nc, dt=_f32):
    return pl.BlockSpec((R, nc), lambda i: (i, 0))


def _degblk(part):
    return pl.BlockSpec((R, 16), lambda i, p=part: (p * NBLK + i, 0))


def _wblk(r, c):
    return pl.BlockSpec((r, c), lambda i: (0, 0))


def _bblk(nc):
    return pl.BlockSpec((nc,), lambda i: (0,))


def _sblk():
    return pl.BlockSpec((1, 1), lambda i: (0, 0), memory_space=pltpu.SMEM)


def _p_specs():
    return [pl.BlockSpec((R, D), lambda i: (i, 0)),
            pl.BlockSpec((R, D), lambda i: (NBLK + i, 0))]


def _hw_shape():
    return jax.ShapeDtypeStruct((NPAD, D), _f32)


def _pro1_call(feat, deg, w):
    return pl.pallas_call(
        _pro1_body,
        grid=(NBLK,),
        in_specs=[_rowblk(D), _degblk(0), _degblk(2), _wblk(D, D)],
        out_specs=_rowblk(D),
        out_shape=_hw_shape(),
    )(feat, deg, deg, w)


def _eppro_call(p, deg, b, w):
    return pl.pallas_call(
        _eppro_body,
        grid=(NBLK,),
        in_specs=_p_specs() + [_degblk(1), _degblk(3), _degblk(0), _degblk(2),
                               _bblk(D), _wblk(D, D)],
        out_specs=_rowblk(D),
        out_shape=_hw_shape(),
    )(p, p, deg, deg, deg, deg, b, w)


def _mid_call(p, deg, b2, wrep, brep, wrec, brec, wd1, eps):
    return pl.pallas_call(
        _mid_body,
        grid=(NBLK,),
        in_specs=_p_specs() + [_degblk(1), _degblk(3), _degblk(0), _degblk(2),
                               _bblk(D), _wblk(D, Z), _bblk(Z), _wblk(Z, D),
                               _bblk(D), _wblk(D, D), _rowblk(Z)],
        out_specs=[_rowblk(D), _sblk()],
        out_shape=[_hw_shape(), jax.ShapeDtypeStruct((1, 1), _f32)],
    )(p, p, deg, deg, deg, deg, b2, wrep, brep, wrec, brec, wd1, eps)


def _final_call(p, deg, b, feat, kl):
    return pl.pallas_call(
        _final_body,
        grid=(NBLK,),
        in_specs=_p_specs() + [_degblk(1), _degblk(3), _bblk(D), _rowblk(D),
                               _sblk()],
        out_specs=_sblk(),
        out_shape=jax.ShapeDtypeStruct((1, 1), _f32),
    )(p, p, deg, deg, b, feat, kl)


def kernel(features, edge_index, enc_W1, enc_b1, enc_W2, enc_b2, W_rep, b_rep,
           W_rec, b_rec, dec_W1, dec_b1, dec_W2, dec_b2):
    pad_idx = N + jnp.arange(E_PAD - E, dtype=jnp.int32) % (NSP - N)
    src3 = jnp.concatenate([edge_index[0], pad_idx]).reshape(NT * CH, K)
    dst3 = jnp.concatenate([edge_index[1], pad_idx]).reshape(NT * CH, K)
    sd3 = jnp.stack([src3, dst3], axis=1)
    feat = jnp.pad(features, ((0, NPAD - N), (0, 0)))
    eps = jnp.asarray(_eps_pad())
    zeros = jnp.zeros((RPTS, D), _f32)
    zeros16 = jnp.zeros((RPT, 16), _f32)
    ones = jnp.ones((K, 16), _f32)

    deg = _deg(src3.reshape(NT, CH, K), dst3.reshape(NT, CH, K),
               zeros16, ones)
    hw1 = _pro1_call(feat, deg, enc_W1)
    p = _spmm(hw1, sd3, zeros)
    hw2 = _eppro_call(p, deg, enc_b1, enc_W2)
    p = _spmm(hw2, sd3, zeros)
    hw3, kl = _mid_call(p, deg, enc_b2, W_rep, b_rep, W_rec, b_rec, dec_W1, eps)
    p = _spmm(hw3, sd3, zeros)
    hw4 = _eppro_call(p, deg, dec_b1, dec_W2)
    p = _spmm(hw4, sd3, zeros)
    loss = _final_call(p, deg, dec_b2, feat, kl)
    return loss[0, 0]
